# Initial kernel scaffold; baseline (speedup 1.0000x reference)
import jax
import jax.numpy as jnp
from jax.experimental import pallas as pl


def kernel(cats, conts, W_cat, b_cat, W_cont, b_cont):
    raise NotImplementedError("write your pallas kernel here")


# speedup vs baseline: 2.3117x; 2.3117x over previous
import functools

import jax
import jax.numpy as jnp
from jax import lax
from jax.experimental import pallas as pl
from jax.experimental.pallas import tpu as pltpu
from jax.experimental.pallas import tpu_sc as plsc

B, NC, NF, C, D = 4096, 26, 13, 1000, 128
NW = 32
RPW = B // NW
CB = 4
CHUNK = CB * NC
NSTEP = RPW // CB
NV = D // 16


def _dense_body(conts_ref, w_ref, bcat_ref, bcont_ref, out_ref):
    conts = conts_ref[...]
    mu = jnp.sum(conts, axis=0, keepdims=True) / B
    cz = conts - mu
    var = jnp.sum(cz * cz, axis=0, keepdims=True) / (B - 1)
    sd = jnp.sqrt(var)
    sd = jnp.where(sd > 0.0, sd, 1.0)
    z = cz / (sd + 1e-8)
    bias = (jnp.sum(bcat_ref[...], axis=0, keepdims=True)
            + jnp.sum(bcont_ref[...], axis=0, keepdims=True))
    out_ref[...] = (
        jnp.dot(z, w_ref[...], preferred_element_type=jnp.float32) + bias)


def _dense_part(conts, w_cont, b_cat, b_cont):
    return pl.pallas_call(
        _dense_body,
        out_shape=jax.ShapeDtypeStruct((B, D), jnp.float32),
    )(conts, w_cont, b_cat, b_cont)


def _sc_body(table_hbm, idx_hbm, dense_hbm, out_hbm,
             idx_v, buf0, buf1, acc_v, sem0, sem1):
    cid = lax.axis_index("c")
    sid = lax.axis_index("s")
    wid = sid * 2 + cid
    base = wid * RPW

    pltpu.sync_copy(idx_hbm.at[pl.ds(wid * NSTEP, NSTEP)], idx_v)
    pltpu.sync_copy(dense_hbm.at[pl.ds(base, RPW)], acc_v)

    bufs = (buf0, buf1)
    sems = (sem0, sem1)
    cps = [None, None]
    cps[0] = pltpu.async_copy(table_hbm.at[idx_v.at[0]], bufs[0], sems[0])

    for j in range(NSTEP):
        p = j & 1
        if j + 1 < NSTEP:
            cps[1 - p] = pltpu.async_copy(
                table_hbm.at[idx_v.at[j + 1]], bufs[1 - p], sems[1 - p])
        cps[p].wait()
        buf = bufs[p]

        def cb_body(cb, _, buf=buf, j=j):
            row = j * CB + cb

            def n_body(n, accs):
                r = cb * NC + n
                return tuple(accs[d] + buf[r, pl.ds(d * 16, 16)]
                             for d in range(NV))

            accs = tuple(acc_v[row, pl.ds(d * 16, 16)] for d in range(NV))
            accs = lax.fori_loop(0, NC, n_body, accs)
            for d in range(NV):
                acc_v[row, pl.ds(d * 16, 16)] = accs[d]
            return 0

        lax.fori_loop(0, CB, cb_body, 0)

    pltpu.sync_copy(acc_v, out_hbm.at[pl.ds(base, RPW)])


def _sc_gather_sum(table, idx2d, dense):
    mesh = plsc.VectorSubcoreMesh(core_axis_name="c", subcore_axis_name="s",
                                  num_cores=2, num_subcores=16)
    f = pl.kernel(
        _sc_body, mesh=mesh,
        out_type=jax.ShapeDtypeStruct((B, D), jnp.float32),
        scratch_types=[
            pltpu.VMEM((NSTEP, CHUNK), jnp.int32),
            pltpu.VMEM((CHUNK, D), jnp.float32),
            pltpu.VMEM((CHUNK, D), jnp.float32),
            pltpu.VMEM((RPW, D), jnp.float32),
            pltpu.SemaphoreType.DMA,
            pltpu.SemaphoreType.DMA,
        ],
    )
    return f(table, idx2d, dense)


def kernel(cats, conts, W_cat, b_cat, W_cont, b_cont):
    dense = _dense_part(conts, W_cont, b_cat, b_cont)
    table = W_cat.reshape(NC * C, D)
    idx = (cats.astype(jnp.int32)
           + (jnp.arange(NC, dtype=jnp.int32) * C)[None, :])
    idx2d = idx.reshape(NW * NSTEP, CHUNK)
    return _sc_gather_sum(table, idx2d, dense)
